# baseline (speedup 1.0000x reference)
import jax
import jax.numpy as jnp
from jax.experimental import pallas as pl

N = 1048576
BLK = 8192
GB = N // BLK


def _volume_kernel(x_ref, y_ref, w1_ref, b1_ref, wd_ref, bd_ref,
                   wc1_ref, wc2_ref, bc_ref, ab_ref,
                   d_ref, c0_ref, c1_ref, c2_ref):
    f32 = jnp.float32
    xt = x_ref[...]
    yt = y_ref[...]
    a0 = ab_ref[:, 0:1]
    a1 = ab_ref[:, 1:2]
    mask = jnp.all((xt >= a0) & (xt <= a1), axis=0, keepdims=True)
    f = jnp.maximum(
        jnp.dot(w1_ref[...], xt, preferred_element_type=f32)
        + b1_ref[...], 0.0)
    dl = jnp.dot(wd_ref[...], f, preferred_element_type=f32) + bd_ref[...]
    dens = jnp.maximum(dl, 0.0) + jnp.log1p(jnp.exp(-jnp.abs(dl)))
    cl = (jnp.dot(wc1_ref[...], f, preferred_element_type=f32)
          + jnp.dot(wc2_ref[...], yt, preferred_element_type=f32)
          + bc_ref[...])
    col = 1.0 / (1.0 + jnp.exp(-cl))
    zero = jnp.float32(0.0)
    dm = jnp.where(mask, dens, zero)
    cm = jnp.where(mask, col, zero)
    j = pl.program_id(0) % 8
    d_ref[pl.ds(j, 1), :] = dm
    c0_ref[pl.ds(j, 1), :] = cm[0:1, :]
    c1_ref[pl.ds(j, 1), :] = cm[1:2, :]
    c2_ref[pl.ds(j, 1), :] = cm[2:3, :]


def kernel(xyz, ynm, W_enc, b_enc, W_d, b_d, W_c, b_c, aabb):
    f32 = jnp.float32
    xt = jnp.stack([xyz[:, 0], xyz[:, 1], xyz[:, 2]], axis=0)
    ybf = ynm.astype(jnp.bfloat16)
    yt = jnp.stack([ybf[:, k] for k in range(16)], axis=0)

    span = aabb[1] - aabb[0]
    s = 2.0 / span
    t = -2.0 * aabb[0] / span - 1.0
    w1t = (s[:, None] * W_enc).T
    b1t = (t @ W_enc + b_enc).reshape(16, 1)

    def _cst(shape):
        return pl.BlockSpec(shape, lambda i: (0, 0))

    def _out():
        return pl.BlockSpec((8, BLK), lambda i: (i // 8, 0))

    out = pl.pallas_call(
        _volume_kernel,
        grid=(GB,),
        in_specs=[
            pl.BlockSpec((3, BLK), lambda i: (0, i)),
            pl.BlockSpec((16, BLK), lambda i: (0, i)),
            _cst((16, 3)),
            _cst((16, 1)),
            _cst((1, 16)),
            _cst((1, 1)),
            _cst((3, 16)),
            _cst((3, 16)),
            _cst((3, 1)),
            _cst((3, 2)),
        ],
        out_specs=[_out(), _out(), _out(), _out()],
        out_shape=[jax.ShapeDtypeStruct((GB, BLK), f32)] * 4,
    )(xt, yt, w1t, b1t, W_d.T, b_d.reshape(1, 1), W_c[:16].T, W_c[16:].T,
      b_c.reshape(3, 1), aabb.T)
    out_d = out[0].reshape(N, 1)
    out_c = jnp.stack([out[1], out[2], out[3]], axis=-1).reshape(N, 3)
    return (out_d, out_c)
